# baseline (speedup 1.0000x reference)
import jax
import jax.numpy as jnp
from jax.experimental import pallas as pl
from jax.experimental.pallas import tpu as pltpu

_ALPHA = 0.2
_NEG = -1e9
_BB = 8


def _gat_kernel(e_ref, x_ref, m_ref, wk_ref, wv_ref, akm_ref, sel_ref,
                eps_ref, o_ref):
    bb, n, d = x_ref.shape
    h_total = akm_ref.shape[1]

    x = x_ref[...].reshape(bb * n, d)
    eps = eps_ref[0, 0]

    kproj = jnp.dot(x, wk_ref[...], preferred_element_type=jnp.float32)
    kproj = jnp.where(kproj >= 0, kproj, _ALPHA * kproj)
    v = jnp.dot(x, wv_ref[...], preferred_element_type=jnp.float32)
    expsk = jnp.exp(jnp.dot(kproj, akm_ref[...],
                            preferred_element_type=jnp.float32))
    w = v * jnp.dot(expsk, sel_ref[...], preferred_element_type=jnp.float32)

    row = jax.lax.broadcasted_iota(jnp.int32, (1, n, n), 1)
    col = jax.lax.broadcasted_iota(jnp.int32, (1, n, n), 2)
    diagf = (row == col).astype(jnp.float32)
    cdiag = (jnp.exp(eps) - 1.0) * diagf
    expe3 = jnp.exp(e_ref[...]) * (jnp.maximum(m_ref[...], diagf) + cdiag)

    s4s = []
    for g in range(bb):
        expe = expe3[g]
        sl = slice(g * n, (g + 1) * n)
        o_ref[g] = jnp.dot(expe, w[sl, :], preferred_element_type=jnp.float32)
        s4s.append(jnp.dot(expe, expsk[sl, :],
                           preferred_element_type=jnp.float32))
    for g in range(bb):
        rbig = jnp.dot(1.0 / s4s[g], sel_ref[...],
                       preferred_element_type=jnp.float32)
        o_ref[g] = o_ref[g] * rbig


def kernel(e, x_atm, m, Wq, Wk, Wv, aq, ak, eps):
    b, n, d = x_atm.shape
    h, _, kd = Wk.shape
    vd = Wv.shape[2]

    wk_f = Wk.transpose(1, 0, 2).reshape(d, h * kd)
    wv_f = Wv.transpose(1, 0, 2).reshape(d, h * vd)
    akm = (ak[:, :, None] * jnp.eye(h, dtype=ak.dtype)[:, None, :]).reshape(h * kd, h)
    sel = jnp.repeat(jnp.eye(h, dtype=jnp.float32), vd, axis=1)
    eps2 = eps.reshape(1, 1)

    bb = _BB
    return pl.pallas_call(
        _gat_kernel,
        grid=(b // bb,),
        in_specs=[
            pl.BlockSpec((bb, n, n), lambda i: (i, 0, 0)),
            pl.BlockSpec((bb, n, d), lambda i: (i, 0, 0)),
            pl.BlockSpec((bb, n, n), lambda i: (i, 0, 0)),
            pl.BlockSpec((d, h * kd), lambda i: (0, 0)),
            pl.BlockSpec((d, h * vd), lambda i: (0, 0)),
            pl.BlockSpec((h * kd, h), lambda i: (0, 0)),
            pl.BlockSpec((h, h * vd), lambda i: (0, 0)),
            pl.BlockSpec((1, 1), lambda i: (0, 0)),
        ],
        out_specs=pl.BlockSpec((bb, n, h * vd), lambda i: (i, 0, 0)),
        out_shape=jax.ShapeDtypeStruct((b, n, h * vd), jnp.float32),
        compiler_params=pltpu.CompilerParams(
            dimension_semantics=("parallel",)),
    )(e, x_atm, m, wk_f, wv_f, akm, sel, eps2)


# speedup vs baseline: 1.4759x; 1.4759x over previous
import jax
import jax.numpy as jnp
from jax.experimental import pallas as pl
from jax.experimental.pallas import tpu as pltpu

_ALPHA = 0.2
_NEG = -1e9
_BB = 32


def _gat_kernel(e_ref, x_ref, m_ref, wk_ref, wv_ref, akm_ref, sel_ref,
                eps_ref, o_ref):
    bb, n, d = x_ref.shape
    h_total = akm_ref.shape[1]

    x = x_ref[...].reshape(bb * n, d)
    eps = eps_ref[0, 0]

    kproj = jnp.dot(x, wk_ref[...], preferred_element_type=jnp.float32)
    kproj = jnp.where(kproj >= 0, kproj, _ALPHA * kproj)
    v = jnp.dot(x, wv_ref[...], preferred_element_type=jnp.float32)
    expsk = jnp.exp(jnp.dot(kproj, akm_ref[...],
                            preferred_element_type=jnp.float32))
    w = v * jnp.dot(expsk, sel_ref[...], preferred_element_type=jnp.float32)

    row = jax.lax.broadcasted_iota(jnp.int32, (1, n, n), 1)
    col = jax.lax.broadcasted_iota(jnp.int32, (1, n, n), 2)
    diagf = (row == col).astype(jnp.float32)
    cdiag = (jnp.exp(eps) - 1.0) * diagf
    expe3 = jnp.exp(e_ref[...]) * (jnp.maximum(m_ref[...], diagf) + cdiag)

    s4s = []
    for g in range(bb):
        expe = expe3[g]
        sl = slice(g * n, (g + 1) * n)
        o_ref[g] = jnp.dot(expe, w[sl, :], preferred_element_type=jnp.float32)
        s4s.append(jnp.dot(expe, expsk[sl, :],
                           preferred_element_type=jnp.float32))
    for g in range(bb):
        rbig = jnp.dot(1.0 / s4s[g], sel_ref[...],
                       preferred_element_type=jnp.float32)
        o_ref[g] = o_ref[g] * rbig


def kernel(e, x_atm, m, Wq, Wk, Wv, aq, ak, eps):
    b, n, d = x_atm.shape
    h, _, kd = Wk.shape
    vd = Wv.shape[2]

    wk_f = Wk.transpose(1, 0, 2).reshape(d, h * kd)
    wv_f = Wv.transpose(1, 0, 2).reshape(d, h * vd)
    akm = (ak[:, :, None] * jnp.eye(h, dtype=ak.dtype)[:, None, :]).reshape(h * kd, h)
    sel = jnp.repeat(jnp.eye(h, dtype=jnp.float32), vd, axis=1)
    eps2 = eps.reshape(1, 1)

    bb = _BB
    return pl.pallas_call(
        _gat_kernel,
        grid=(b // bb,),
        in_specs=[
            pl.BlockSpec((bb, n, n), lambda i: (i, 0, 0)),
            pl.BlockSpec((bb, n, d), lambda i: (i, 0, 0)),
            pl.BlockSpec((bb, n, n), lambda i: (i, 0, 0)),
            pl.BlockSpec((d, h * kd), lambda i: (0, 0)),
            pl.BlockSpec((d, h * vd), lambda i: (0, 0)),
            pl.BlockSpec((h * kd, h), lambda i: (0, 0)),
            pl.BlockSpec((h, h * vd), lambda i: (0, 0)),
            pl.BlockSpec((1, 1), lambda i: (0, 0)),
        ],
        out_specs=pl.BlockSpec((bb, n, h * vd), lambda i: (i, 0, 0)),
        out_shape=jax.ShapeDtypeStruct((b, n, h * vd), jnp.float32),
        compiler_params=pltpu.CompilerParams(
            dimension_semantics=("parallel",)),
    )(e, x_atm, m, wk_f, wv_f, akm, sel, eps2)


# speedup vs baseline: 1.4973x; 1.0145x over previous
import jax
import jax.numpy as jnp
from jax.experimental import pallas as pl
from jax.experimental.pallas import tpu as pltpu

_ALPHA = 0.2
_NEG = -1e9
_BB = 64


def _gat_kernel(e_ref, x_ref, m_ref, wk_ref, wv_ref, akm_ref, sel_ref,
                eps_ref, o_ref):
    bb, n, d = x_ref.shape
    h_total = akm_ref.shape[1]

    x = x_ref[...].reshape(bb * n, d)
    eps = eps_ref[0, 0]

    kproj = jnp.dot(x, wk_ref[...], preferred_element_type=jnp.float32)
    kproj = jnp.where(kproj >= 0, kproj, _ALPHA * kproj)
    v = jnp.dot(x, wv_ref[...], preferred_element_type=jnp.float32)
    expsk = jnp.exp(jnp.dot(kproj, akm_ref[...],
                            preferred_element_type=jnp.float32))
    w = v * jnp.dot(expsk, sel_ref[...], preferred_element_type=jnp.float32)

    row = jax.lax.broadcasted_iota(jnp.int32, (1, n, n), 1)
    col = jax.lax.broadcasted_iota(jnp.int32, (1, n, n), 2)
    diagf = (row == col).astype(jnp.float32)
    cdiag = (jnp.exp(eps) - 1.0) * diagf
    expe3 = jnp.exp(e_ref[...]) * (jnp.maximum(m_ref[...], diagf) + cdiag)

    s4s = []
    for g in range(bb):
        expe = expe3[g]
        sl = slice(g * n, (g + 1) * n)
        o_ref[g] = jnp.dot(expe, w[sl, :], preferred_element_type=jnp.float32)
        s4s.append(jnp.dot(expe, expsk[sl, :],
                           preferred_element_type=jnp.float32))
    for g in range(bb):
        rbig = jnp.dot(1.0 / s4s[g], sel_ref[...],
                       preferred_element_type=jnp.float32)
        o_ref[g] = o_ref[g] * rbig


def kernel(e, x_atm, m, Wq, Wk, Wv, aq, ak, eps):
    b, n, d = x_atm.shape
    h, _, kd = Wk.shape
    vd = Wv.shape[2]

    wk_f = Wk.transpose(1, 0, 2).reshape(d, h * kd)
    wv_f = Wv.transpose(1, 0, 2).reshape(d, h * vd)
    akm = (ak[:, :, None] * jnp.eye(h, dtype=ak.dtype)[:, None, :]).reshape(h * kd, h)
    sel = jnp.repeat(jnp.eye(h, dtype=jnp.float32), vd, axis=1)
    eps2 = eps.reshape(1, 1)

    bb = _BB
    return pl.pallas_call(
        _gat_kernel,
        grid=(b // bb,),
        in_specs=[
            pl.BlockSpec((bb, n, n), lambda i: (i, 0, 0)),
            pl.BlockSpec((bb, n, d), lambda i: (i, 0, 0)),
            pl.BlockSpec((bb, n, n), lambda i: (i, 0, 0)),
            pl.BlockSpec((d, h * kd), lambda i: (0, 0)),
            pl.BlockSpec((d, h * vd), lambda i: (0, 0)),
            pl.BlockSpec((h * kd, h), lambda i: (0, 0)),
            pl.BlockSpec((h, h * vd), lambda i: (0, 0)),
            pl.BlockSpec((1, 1), lambda i: (0, 0)),
        ],
        out_specs=pl.BlockSpec((bb, n, h * vd), lambda i: (i, 0, 0)),
        out_shape=jax.ShapeDtypeStruct((b, n, h * vd), jnp.float32),
        compiler_params=pltpu.CompilerParams(
            dimension_semantics=("parallel",)),
    )(e, x_atm, m, wk_f, wv_f, akm, sel, eps2)
